# Initial kernel scaffold; baseline (speedup 1.0000x reference)
import jax
import jax.numpy as jnp
from jax.experimental import pallas as pl


def kernel(inputs, rot):
    raise NotImplementedError("write your pallas kernel here")


# speedup vs baseline: 4.6589x; 4.6589x over previous
import jax
import jax.numpy as jnp
from jax import lax
from jax.experimental import pallas as pl
from jax.experimental.pallas import tpu as pltpu

N = 2000
D = 256
S = 500
C = 4
K = 5
DM = 0.1
RB = 200
CSB = 250

_HI = lax.Precision.HIGHEST


def _compute_body(points_ref, rot_ref, cv_ref):
    p = points_ref[0]
    r2 = rot_ref[...]

    mul = jnp.dot(p, r2, preferred_element_type=jnp.float32)
    cmul = jnp.concatenate([mul, -mul], axis=1)
    best = cmul[:, 0:1]
    binv = jnp.zeros((N, 1), jnp.float32)
    for j in range(1, 2 * (C // 2)):
        cj = cmul[:, j:j + 1]
        gt = cj > best
        binv = jnp.where(gt, float(j), binv)
        best = jnp.where(gt, cj, best)

    row_iota = lax.broadcasted_iota(jnp.int32, (N, 1), 0).astype(jnp.float32)

    lane4 = lax.broadcasted_iota(jnp.int32, (N, C), 1).astype(jnp.float32)
    oh = (binv == lane4).astype(jnp.float32)
    se_blocks = []
    for b in range(N // CSB):
        rb = (lax.broadcasted_iota(jnp.int32, (CSB, N), 0) + b * CSB)
        cb = lax.broadcasted_iota(jnp.int32, (CSB, N), 1)
        Lb = (cb < rb).astype(jnp.float32)
        se_blocks.append(lax.dot_general(
            Lb, oh, (((1,), (0,)), ((), ())), precision=_HI))
    S_excl = jnp.concatenate(se_blocks, axis=0)
    cnt = jnp.sum(oh, axis=0, keepdims=True)
    r4 = lax.broadcasted_iota(jnp.int32, (C, C), 0)
    c4 = lax.broadcasted_iota(jnp.int32, (C, C), 1)
    U4 = (r4 < c4).astype(jnp.float32)
    off = lax.dot_general(cnt, U4, (((1,), (0,)), ((), ())),
                          precision=_HI)
    rank = (jnp.sum(oh * off, axis=1, keepdims=True)
            + jnp.sum(oh * S_excl, axis=1, keepdims=True))

    na_g = jnp.sum(p * p, axis=1, keepdims=True)

    lane_ss = lax.broadcasted_iota(jnp.int32, (S, S), 1).astype(jnp.float32)
    lane_ns = lax.broadcasted_iota(jnp.int32, (N, S), 1).astype(jnp.float32)

    def chunk_step(c, cv_acc):
        cf = c.astype(jnp.float32) * float(S)
        Hc = (rank == (cf + lane_ns)).astype(jnp.float32)

        parts = lax.dot_general(Hc, p, (((0,), (0,)), ((), ())),
                                precision=_HI)
        na_col = jnp.sum(parts * parts, axis=1, keepdims=True)
        na_row = lax.dot_general(na_g, Hc, (((0,), (0,)), ((), ())),
                                 precision=_HI)
        gidx_row = lax.dot_general(row_iota, Hc, (((0,), (0,)), ((), ())),
                                   precision=_HI)

        dcc = lax.dot_general(parts, parts, (((1,), (1,)), ((), ())))
        dm2 = na_col - 2.0 * dcc + na_row
        E = jnp.exp(-DM * jnp.sqrt(jnp.maximum(dm2, 1e-6)))

        work = E
        vloc, cloc = [], []
        for k in range(K):
            mx = jnp.max(work, axis=1, keepdims=True)
            ismx = work == mx
            am = jnp.min(jnp.where(ismx, lane_ss, float(2 * N)),
                         axis=1, keepdims=True)
            sel = lane_ss == am
            gcol = jnp.sum(jnp.where(sel, gidx_row, 0.0),
                           axis=1, keepdims=True)
            vloc.append(mx)
            cloc.append(gcol)
            work = jnp.where(sel, -1.0, work)

        cvc = jnp.concatenate(vloc + cloc, axis=1)
        return cv_acc + lax.dot_general(
            Hc, cvc, (((1,), (0,)), ((), ())), precision=_HI)

    cv_ref[0] = lax.fori_loop(0, C, chunk_step,
                              jnp.zeros((N, 2 * K), jnp.float32))


def _assemble_body(cv_ref, out_ref):
    cv = cv_ref[0]
    ci = lax.broadcasted_iota(jnp.int32, (RB, N), 1).astype(jnp.float32)
    acc = jnp.where(ci == cv[:, K:K + 1], cv[:, 0:1], 0.0)
    for k in range(1, K):
        acc = acc + jnp.where(ci == cv[:, K + k:K + k + 1],
                              cv[:, k:k + 1], 0.0)
    out_ref[0] = acc


def kernel(inputs, rot):
    B = inputs.shape[0]
    rot2 = rot[:, : C // 2]
    cv = pl.pallas_call(
        _compute_body,
        grid=(B,),
        in_specs=[
            pl.BlockSpec((1, N, D), lambda b: (b, 0, 0)),
            pl.BlockSpec((D, C // 2), lambda b: (0, 0)),
        ],
        out_specs=pl.BlockSpec((1, N, 2 * K), lambda b: (b, 0, 0)),
        out_shape=jax.ShapeDtypeStruct((B, N, 2 * K), jnp.float32),
    )(inputs, rot2)
    out = pl.pallas_call(
        _assemble_body,
        grid=(B, N // RB),
        in_specs=[pl.BlockSpec((1, RB, 2 * K), lambda b, r: (b, r, 0))],
        out_specs=pl.BlockSpec((1, RB, N), lambda b, r: (b, r, 0)),
        out_shape=jax.ShapeDtypeStruct((B, N, N), jnp.float32),
    )(cv)
    return out
